# baseline (speedup 1.0000x reference)
import functools

import jax
import jax.numpy as jnp
from jax import lax
from jax.experimental import pallas as pl
from jax.experimental.pallas import tpu as pltpu
from jax.experimental.pallas import tpu_sc as plsc

N = 10000
E = 320000
F = 128
C = 16

NC = 2
NS = 16
NW = NC * NS
EPT = E // NW
K = 80
NCH = EPT // K
PCH = 63
ROWS_SP = 10112
WR = ROWS_SP // NS
DEG_P = 10240
DEG_S = DEG_P // NS

_mesh = plsc.VectorSubcoreMesh(
    core_axis_name="c", subcore_axis_name="s", num_cores=NC, num_subcores=NS
)


def _fill_zero_2d(zbuf):
  z16 = jnp.zeros((16,), jnp.float32)
  for r in range(zbuf.shape[0]):
    for k in range(zbuf.shape[1] // 16):
      zbuf[r, pl.ds(k * 16, 16)] = z16


def _fill_zero_1d(zvec):
  z16 = jnp.zeros((16,), jnp.float32)
  for k in range(zvec.shape[0] // 16):
    zvec[pl.ds(k * 16, 16)] = z16


def _load_idx_2d(src_hbm, base, buf2d, sem, n=NCH):
  descs = []
  for j in range(n):
    if j >= 64:
      descs[j - 64].wait()
    descs.append(
        pltpu.async_copy(src_hbm.at[pl.ds(base + j * K, K)], buf2d.at[j], sem))
  for d in descs[max(0, n - 64):]:
    d.wait()


def _zero_spmem_rows(zbuf, sp, row0, nrows, sem):
  zr = zbuf.shape[0]
  full, rem = nrows // zr, nrows % zr
  descs = []
  for t in range(full):
    descs.append(
        pltpu.async_copy(zbuf, sp.at[pl.ds(row0 + t * zr, zr)], sem))
  if rem:
    descs.append(
        pltpu.async_copy(zbuf.at[pl.ds(0, rem)],
                         sp.at[pl.ds(row0 + full * zr, rem)], sem))
  for d in descs:
    d.wait()


@functools.partial(
    pl.kernel,
    out_type=(
        jax.ShapeDtypeStruct((NC * DEG_P,), jnp.float32),
        jax.ShapeDtypeStruct((NC * DEG_P,), jnp.float32),
    ),
    mesh=_mesh,
    scratch_types=[
        pltpu.VMEM((NCH, K), jnp.int32),
        pltpu.VMEM((NCH, K), jnp.int32),
        pltpu.VMEM((K,), jnp.float32),
        pltpu.VMEM((DEG_S,), jnp.float32),
        pltpu.VMEM_SHARED((DEG_P,), jnp.float32),
        pltpu.VMEM_SHARED((DEG_P,), jnp.float32),
        pltpu.SemaphoreType.DMA,
        pltpu.SemaphoreType.DMA,
    ],
)
def _sc_degrees(ei, dop, dip, src2d, dst2d, ones, zvec,
                do_sp, di_sp, sem, sem2):
  c = lax.axis_index("c")
  s = lax.axis_index("s")
  wid = s * NC + c
  base = wid * EPT

  one16 = jnp.full((16,), 1.0, jnp.float32)
  for k in range(K // 16):
    ones[pl.ds(k * 16, 16)] = one16
  _fill_zero_1d(zvec)

  _load_idx_2d(ei, base, src2d, sem)
  _load_idx_2d(ei, E + base, dst2d, sem)

  pltpu.sync_copy(zvec, do_sp.at[pl.ds(s * DEG_S, DEG_S)])
  pltpu.sync_copy(zvec, di_sp.at[pl.ds(s * DEG_S, DEG_S)])
  plsc.subcore_barrier()

  descs = []
  for j in range(NCH):
    if j >= 8:
      descs[j - 8][0].wait()
      descs[j - 8][1].wait()
    descs.append((
        pltpu.async_copy(ones, do_sp.at[src2d.at[j]], sem, add=True),
        pltpu.async_copy(ones, di_sp.at[dst2d.at[j]], sem2, add=True),
    ))
  for p in descs[max(0, NCH - 8):]:
    p[0].wait()
    p[1].wait()

  plsc.subcore_barrier()
  off = pl.multiple_of(c * DEG_P + s * DEG_S, 128)
  pltpu.sync_copy(do_sp.at[pl.ds(s * DEG_S, DEG_S)], dop.at[pl.ds(off, DEG_S)])
  pltpu.sync_copy(di_sp.at[pl.ds(s * DEG_S, DEG_S)], dip.at[pl.ds(off, DEG_S)])


def _make_sc_agg(with_c):
  out_type = [jax.ShapeDtypeStruct((NC, ROWS_SP, F), jnp.float32)]
  scratch = [
      pltpu.VMEM((PCH, K), jnp.int32),
      pltpu.VMEM((PCH, K), jnp.int32),
      pltpu.VMEM((2, K, F), jnp.float32),
      pltpu.VMEM((40, F), jnp.float32),
      pltpu.VMEM_SHARED((ROWS_SP, F), jnp.float32),
      pltpu.SemaphoreType.DMA,
      pltpu.SemaphoreType.DMA,
  ]
  if with_c:
    out_type.append(jax.ShapeDtypeStruct((NC * DEG_P,), jnp.float32))
    scratch += [
        pltpu.VMEM((2, K), jnp.float32),
        pltpu.VMEM((DEG_S,), jnp.float32),
        pltpu.VMEM_SHARED((DEG_P,), jnp.float32),
        pltpu.SemaphoreType.DMA,
        pltpu.SemaphoreType.DMA,
    ]

  def body(h, ei, *refs):
    if with_c:
      (sin, aggp, cp, src2d, dst2d, rows, zbuf, agg_sp, gsem, ssem,
       vals, zvec, c_sp, cgsem, cssem) = refs
    else:
      (aggp, src2d, dst2d, rows, zbuf, agg_sp, gsem, ssem) = refs
    c = lax.axis_index("c")
    s = lax.axis_index("s")
    wid = s * NC + c
    base = wid * EPT

    _fill_zero_2d(zbuf)
    _zero_spmem_rows(zbuf, agg_sp, s * WR, WR, gsem)

    if with_c:
      _fill_zero_1d(zvec)
      pltpu.sync_copy(zvec, c_sp.at[pl.ds(s * DEG_S, DEG_S)])

    plsc.subcore_barrier()

    for jlo in (0, PCH):
      n = min(NCH - jlo, PCH)
      _load_idx_2d(ei, base + jlo * K, src2d, gsem, n)
      _load_idx_2d(ei, E + base + jlo * K, dst2d, gsem, n)

      ga = [None] * n
      sa = [None] * n
      gc = [None] * n
      sc = [None] * n
      ga[0] = pltpu.async_copy(h.at[src2d.at[0]], rows.at[0], gsem)
      if with_c:
        gc[0] = pltpu.async_copy(sin.at[dst2d.at[0]], vals.at[0], cgsem)
      for j in range(n):
        ga[j].wait()
        sa[j] = pltpu.async_copy(
            rows.at[j % 2], agg_sp.at[dst2d.at[j]], ssem, add=True)
        if with_c:
          gc[j].wait()
          sc[j] = pltpu.async_copy(
              vals.at[j % 2], c_sp.at[src2d.at[j]], cssem, add=True)
        if j + 1 < n:
          if j >= 1:
            sa[j - 1].wait()
            if with_c:
              sc[j - 1].wait()
          ga[j + 1] = pltpu.async_copy(
              h.at[src2d.at[j + 1]], rows.at[(j + 1) % 2], gsem)
          if with_c:
            gc[j + 1] = pltpu.async_copy(
                sin.at[dst2d.at[j + 1]], vals.at[(j + 1) % 2], cgsem)
      sa[n - 2].wait()
      sa[n - 1].wait()
      if with_c:
        sc[n - 2].wait()
        sc[n - 1].wait()

    plsc.subcore_barrier()
    pltpu.sync_copy(agg_sp.at[pl.ds(s * WR, WR)],
                    aggp.at[c, pl.ds(s * WR, WR)])
    if with_c:
      off = pl.multiple_of(c * DEG_P + s * DEG_S, 128)
      pltpu.sync_copy(c_sp.at[pl.ds(s * DEG_S, DEG_S)],
                      cp.at[pl.ds(off, DEG_S)])

  return pl.kernel(body, out_type=tuple(out_type) if with_c else out_type[0],
                   mesh=_mesh, scratch_types=scratch)


_sc_agg_c = _make_sc_agg(True)
_sc_agg = _make_sc_agg(False)


R = 5000
GRID = N // R


def _tc_mm_body(x_ref, w_ref, h_ref):
  h_ref[...] = jnp.dot(x_ref[...], w_ref[...],
                       preferred_element_type=jnp.float32)


def _tc_mm(x, w1):
  return pl.pallas_call(
      _tc_mm_body,
      grid=(GRID,),
      in_specs=[
          pl.BlockSpec((R, F), lambda i: (i, 0)),
          pl.BlockSpec((F, F), lambda i: (0, 0)),
      ],
      out_specs=pl.BlockSpec((R, F), lambda i: (i, 0)),
      out_shape=jax.ShapeDtypeStruct((N, F), jnp.float32),
  )(x, w1)


def _tc_scale_body(h_ref, aux_ref, out_ref, sin_ref):
  a = aux_ref[...]
  so = lax.rsqrt(jnp.maximum(a[:, 0:1] + a[:, 1:2], 1.0))
  sin_ref[...] = lax.rsqrt(jnp.maximum(a[:, 2:3] + a[:, 3:4], 1.0))
  out_ref[...] = h_ref[...] * so


def _tc_scale(h, aux):
  return pl.pallas_call(
      _tc_scale_body,
      grid=(GRID,),
      in_specs=[
          pl.BlockSpec((R, F), lambda i: (i, 0)),
          pl.BlockSpec((R, 4), lambda i: (i, 0)),
      ],
      out_specs=[
          pl.BlockSpec((R, F), lambda i: (i, 0)),
          pl.BlockSpec((R, 1), lambda i: (i, 0)),
      ],
      out_shape=[
          jax.ShapeDtypeStruct((N, F), jnp.float32),
          jax.ShapeDtypeStruct((N, 1), jnp.float32),
      ],
  )(h, aux)


def _tc_mid_body(aggp_ref, aux_ref, b_ref, w_ref, out_ref):
  a = aux_ref[...]
  so = lax.rsqrt(jnp.maximum(a[:, 0:1] + a[:, 1:2], 1.0))
  si = lax.rsqrt(jnp.maximum(a[:, 2:3] + a[:, 3:4], 1.0))
  agg = aggp_ref[0] + aggp_ref[1]
  h = jnp.maximum(agg * si + b_ref[...], 0.0)
  out_ref[...] = jnp.dot(h * so, w_ref[...],
                         preferred_element_type=jnp.float32)


def _tc_mid(aggp, aux, b, w):
  return pl.pallas_call(
      _tc_mid_body,
      grid=(GRID,),
      in_specs=[
          pl.BlockSpec((NC, R, F), lambda i: (0, i, 0)),
          pl.BlockSpec((R, 4), lambda i: (i, 0)),
          pl.BlockSpec((1, F), lambda i: (0, 0)),
          pl.BlockSpec((F, F), lambda i: (0, 0)),
      ],
      out_specs=pl.BlockSpec((R, F), lambda i: (i, 0)),
      out_shape=jax.ShapeDtypeStruct((N, F), jnp.float32),
  )(aggp, aux, b.reshape(1, F), w)


def _tc_final_body(aggp_ref, aux_ref, cpt_ref, b3_ref, w4_ref, b4_ref,
                   out_ref, acc_ref):
  i = pl.program_id(0)
  a = aux_ref[...]
  so = lax.rsqrt(jnp.maximum(a[:, 0:1] + a[:, 1:2], 1.0))
  si = lax.rsqrt(jnp.maximum(a[:, 2:3] + a[:, 3:4], 1.0))
  cpt = cpt_ref[...]
  w_col = (cpt[:, 0:1] + cpt[:, 1:2]) * so
  agg = aggp_ref[0] + aggp_ref[1]
  h = jnp.maximum(agg * si + b3_ref[...], 0.0)
  part = jnp.sum(h * w_col, axis=0, keepdims=True)

  @pl.when(i == 0)
  def _():
    acc_ref[...] = part

  @pl.when(i > 0)
  def _():
    acc_ref[...] += part

  @pl.when(i == GRID - 1)
  def _():
    out_ref[...] = (
        jnp.dot(acc_ref[...], w4_ref[...], preferred_element_type=jnp.float32)
        / float(N) + b4_ref[...])


def _tc_final(aggp, aux, cpt, b3, w4, b4):
  return pl.pallas_call(
      _tc_final_body,
      grid=(GRID,),
      in_specs=[
          pl.BlockSpec((NC, R, F), lambda i: (0, i, 0)),
          pl.BlockSpec((R, 4), lambda i: (i, 0)),
          pl.BlockSpec((R, 2), lambda i: (i, 0)),
          pl.BlockSpec((1, F), lambda i: (0, 0)),
          pl.BlockSpec((F, C), lambda i: (0, 0)),
          pl.BlockSpec((1, C), lambda i: (0, 0)),
      ],
      out_specs=pl.BlockSpec((1, C), lambda i: (0, 0)),
      out_shape=jax.ShapeDtypeStruct((1, C), jnp.float32),
      scratch_shapes=[pltpu.VMEM((1, F), jnp.float32)],
  )(aggp, aux, cpt, b3.reshape(1, F), w4, b4.reshape(1, C))


def kernel(in_feat, edge_index, W1, b1, W2, b2, W3, b3, W4, b4):
  ei = edge_index.astype(jnp.int32).reshape(2 * E)

  h1u = _tc_mm(in_feat, W1)
  dop, dip = _sc_degrees(ei)
  dop = dop.reshape(NC, DEG_P)
  dip = dip.reshape(NC, DEG_P)
  aux = jnp.concatenate([dop, dip], axis=0)[:, :N].T

  h1p, sin_col = _tc_scale(h1u, aux)
  aggp1, cp = _sc_agg_c(h1p, ei, sin_col.reshape(N))
  h2p = _tc_mid(aggp1, aux, b1, W2)
  aggp2 = _sc_agg(h2p, ei)
  h3p = _tc_mid(aggp2, aux, b2, W3)
  aggp3 = _sc_agg(h3p, ei)
  cpt = cp.reshape(NC, DEG_P)[:, :N].T
  out = _tc_final(aggp3, aux, cpt, b3, W4, b4)
  return out.reshape(C)


# speedup vs baseline: 1.2434x; 1.2434x over previous
import functools

import jax
import jax.numpy as jnp
from jax import lax
from jax.experimental import pallas as pl
from jax.experimental.pallas import tpu as pltpu
from jax.experimental.pallas import tpu_sc as plsc

N = 10000
E = 320000
F = 128
C = 16

NC = 2
NS = 16
NW = NC * NS
EPT = E // NW
K = 80
NCH = EPT // K
PCH = 63
ROWS_SP = 10112
WR = ROWS_SP // NS
DEG_P = 10240
DEG_S = DEG_P // NS

_mesh = plsc.VectorSubcoreMesh(
    core_axis_name="c", subcore_axis_name="s", num_cores=NC, num_subcores=NS
)


def _fill_zero_2d(zbuf):
  z16 = jnp.zeros((16,), jnp.float32)
  for r in range(zbuf.shape[0]):
    for k in range(zbuf.shape[1] // 16):
      zbuf[r, pl.ds(k * 16, 16)] = z16


def _fill_zero_1d(zvec):
  z16 = jnp.zeros((16,), jnp.float32)
  for k in range(zvec.shape[0] // 16):
    zvec[pl.ds(k * 16, 16)] = z16


def _load_idx_2d(src_hbm, base, buf2d, sem, n=NCH):
  descs = []
  for j in range(n):
    if j >= 64:
      descs[j - 64].wait()
    descs.append(
        pltpu.async_copy(src_hbm.at[pl.ds(base + j * K, K)], buf2d.at[j], sem))
  for d in descs[max(0, n - 64):]:
    d.wait()


def _zero_spmem_rows(zbuf, sp, row0, nrows, sem):
  zr = zbuf.shape[0]
  full, rem = nrows // zr, nrows % zr
  descs = []
  for t in range(full):
    descs.append(
        pltpu.async_copy(zbuf, sp.at[pl.ds(row0 + t * zr, zr)], sem))
  if rem:
    descs.append(
        pltpu.async_copy(zbuf.at[pl.ds(0, rem)],
                         sp.at[pl.ds(row0 + full * zr, rem)], sem))
  for d in descs:
    d.wait()


@functools.partial(
    pl.kernel,
    out_type=(
        jax.ShapeDtypeStruct((NC * DEG_P,), jnp.float32),
        jax.ShapeDtypeStruct((NC * DEG_P,), jnp.float32),
    ),
    mesh=_mesh,
    scratch_types=[
        pltpu.VMEM((NCH, K), jnp.int32),
        pltpu.VMEM((NCH, K), jnp.int32),
        pltpu.VMEM((K,), jnp.float32),
        pltpu.VMEM((DEG_S,), jnp.float32),
        pltpu.VMEM_SHARED((DEG_P,), jnp.float32),
        pltpu.VMEM_SHARED((DEG_P,), jnp.float32),
        pltpu.SemaphoreType.DMA,
        pltpu.SemaphoreType.DMA,
    ],
)
def _sc_degrees(ei, dop, dip, src2d, dst2d, ones, zvec,
                do_sp, di_sp, sem, sem2):
  c = lax.axis_index("c")
  s = lax.axis_index("s")
  wid = s * NC + c
  base = wid * EPT

  one16 = jnp.full((16,), 1.0, jnp.float32)
  for k in range(K // 16):
    ones[pl.ds(k * 16, 16)] = one16
  _fill_zero_1d(zvec)

  _load_idx_2d(ei, base, src2d, sem)
  _load_idx_2d(ei, E + base, dst2d, sem)

  pltpu.sync_copy(zvec, do_sp.at[pl.ds(s * DEG_S, DEG_S)])
  pltpu.sync_copy(zvec, di_sp.at[pl.ds(s * DEG_S, DEG_S)])
  plsc.subcore_barrier()

  descs = []
  for j in range(NCH):
    if j >= 8:
      descs[j - 8][0].wait()
      descs[j - 8][1].wait()
    descs.append((
        pltpu.async_copy(ones, do_sp.at[src2d.at[j]], sem, add=True),
        pltpu.async_copy(ones, di_sp.at[dst2d.at[j]], sem2, add=True),
    ))
  for p in descs[max(0, NCH - 8):]:
    p[0].wait()
    p[1].wait()

  plsc.subcore_barrier()
  off = pl.multiple_of(c * DEG_P + s * DEG_S, 128)
  pltpu.sync_copy(do_sp.at[pl.ds(s * DEG_S, DEG_S)], dop.at[pl.ds(off, DEG_S)])
  pltpu.sync_copy(di_sp.at[pl.ds(s * DEG_S, DEG_S)], dip.at[pl.ds(off, DEG_S)])


def _make_sc_agg(with_c):
  out_type = [jax.ShapeDtypeStruct((NC, ROWS_SP, F), jnp.float32)]
  scratch = [
      pltpu.VMEM((PCH, K), jnp.int32),
      pltpu.VMEM((PCH, K), jnp.int32),
      pltpu.VMEM((2, K, F), jnp.float32),
      pltpu.VMEM((40, F), jnp.float32),
      pltpu.VMEM_SHARED((ROWS_SP, F), jnp.float32),
      pltpu.SemaphoreType.DMA,
      pltpu.SemaphoreType.DMA,
  ]
  if with_c:
    out_type.append(jax.ShapeDtypeStruct((NC * DEG_P,), jnp.float32))
    scratch += [
        pltpu.VMEM((2, K), jnp.float32),
        pltpu.VMEM((DEG_S,), jnp.float32),
        pltpu.VMEM_SHARED((DEG_P,), jnp.float32),
        pltpu.SemaphoreType.DMA,
        pltpu.SemaphoreType.DMA,
    ]

  def body(h, ei, *refs):
    if with_c:
      (sin, aggp, cp, src2d, dst2d, rows, zbuf, agg_sp, gsem, ssem,
       vals, zvec, c_sp, cgsem, cssem) = refs
    else:
      (aggp, src2d, dst2d, rows, zbuf, agg_sp, gsem, ssem) = refs
    c = lax.axis_index("c")
    s = lax.axis_index("s")
    wid = s * NC + c
    base = wid * EPT

    _fill_zero_2d(zbuf)
    _zero_spmem_rows(zbuf, agg_sp, s * WR, WR, gsem)

    if with_c:
      _fill_zero_1d(zvec)
      pltpu.sync_copy(zvec, c_sp.at[pl.ds(s * DEG_S, DEG_S)])

    plsc.subcore_barrier()

    for jlo in (0, PCH):
      n = min(NCH - jlo, PCH)
      _load_idx_2d(ei, base + jlo * K, src2d, gsem, n)
      _load_idx_2d(ei, E + base + jlo * K, dst2d, gsem, n)

      ga = [None] * n
      sa = [None] * n
      gc = [None] * n
      sc = [None] * n
      ga[0] = pltpu.async_copy(h.at[src2d.at[0]], rows.at[0], gsem)
      if with_c:
        gc[0] = pltpu.async_copy(sin.at[dst2d.at[0]], vals.at[0], cgsem)
      for j in range(n):
        if j + 1 < n:
          if j >= 1:
            sa[j - 1].wait()
            if with_c:
              sc[j - 1].wait()
          ga[j + 1] = pltpu.async_copy(
              h.at[src2d.at[j + 1]], rows.at[(j + 1) % 2], gsem)
          if with_c:
            gc[j + 1] = pltpu.async_copy(
                sin.at[dst2d.at[j + 1]], vals.at[(j + 1) % 2], cgsem)
        ga[j].wait()
        sa[j] = pltpu.async_copy(
            rows.at[j % 2], agg_sp.at[dst2d.at[j]], ssem, add=True)
        if with_c:
          gc[j].wait()
          sc[j] = pltpu.async_copy(
              vals.at[j % 2], c_sp.at[src2d.at[j]], cssem, add=True)
      sa[n - 2].wait()
      sa[n - 1].wait()
      if with_c:
        sc[n - 2].wait()
        sc[n - 1].wait()

    plsc.subcore_barrier()
    pltpu.sync_copy(agg_sp.at[pl.ds(s * WR, WR)],
                    aggp.at[c, pl.ds(s * WR, WR)])
    if with_c:
      off = pl.multiple_of(c * DEG_P + s * DEG_S, 128)
      pltpu.sync_copy(c_sp.at[pl.ds(s * DEG_S, DEG_S)],
                      cp.at[pl.ds(off, DEG_S)])

  return pl.kernel(body, out_type=tuple(out_type) if with_c else out_type[0],
                   mesh=_mesh, scratch_types=scratch)


_sc_agg_c = _make_sc_agg(True)
_sc_agg = _make_sc_agg(False)


R = 5000
GRID = N // R


def _tc_mm_body(x_ref, w_ref, h_ref):
  h_ref[...] = jnp.dot(x_ref[...], w_ref[...],
                       preferred_element_type=jnp.float32)


def _tc_mm(x, w1):
  return pl.pallas_call(
      _tc_mm_body,
      grid=(GRID,),
      in_specs=[
          pl.BlockSpec((R, F), lambda i: (i, 0)),
          pl.BlockSpec((F, F), lambda i: (0, 0)),
      ],
      out_specs=pl.BlockSpec((R, F), lambda i: (i, 0)),
      out_shape=jax.ShapeDtypeStruct((N, F), jnp.float32),
  )(x, w1)


def _tc_scale_body(h_ref, aux_ref, out_ref, sin_ref):
  a = aux_ref[...]
  so = lax.rsqrt(jnp.maximum(a[:, 0:1] + a[:, 1:2], 1.0))
  sin_ref[...] = lax.rsqrt(jnp.maximum(a[:, 2:3] + a[:, 3:4], 1.0))
  out_ref[...] = h_ref[...] * so


def _tc_scale(h, aux):
  return pl.pallas_call(
      _tc_scale_body,
      grid=(GRID,),
      in_specs=[
          pl.BlockSpec((R, F), lambda i: (i, 0)),
          pl.BlockSpec((R, 4), lambda i: (i, 0)),
      ],
      out_specs=[
          pl.BlockSpec((R, F), lambda i: (i, 0)),
          pl.BlockSpec((R, 1), lambda i: (i, 0)),
      ],
      out_shape=[
          jax.ShapeDtypeStruct((N, F), jnp.float32),
          jax.ShapeDtypeStruct((N, 1), jnp.float32),
      ],
  )(h, aux)


def _tc_mid_body(aggp_ref, aux_ref, b_ref, w_ref, out_ref):
  a = aux_ref[...]
  so = lax.rsqrt(jnp.maximum(a[:, 0:1] + a[:, 1:2], 1.0))
  si = lax.rsqrt(jnp.maximum(a[:, 2:3] + a[:, 3:4], 1.0))
  agg = aggp_ref[0] + aggp_ref[1]
  h = jnp.maximum(agg * si + b_ref[...], 0.0)
  out_ref[...] = jnp.dot(h * so, w_ref[...],
                         preferred_element_type=jnp.float32)


def _tc_mid(aggp, aux, b, w):
  return pl.pallas_call(
      _tc_mid_body,
      grid=(GRID,),
      in_specs=[
          pl.BlockSpec((NC, R, F), lambda i: (0, i, 0)),
          pl.BlockSpec((R, 4), lambda i: (i, 0)),
          pl.BlockSpec((1, F), lambda i: (0, 0)),
          pl.BlockSpec((F, F), lambda i: (0, 0)),
      ],
      out_specs=pl.BlockSpec((R, F), lambda i: (i, 0)),
      out_shape=jax.ShapeDtypeStruct((N, F), jnp.float32),
  )(aggp, aux, b.reshape(1, F), w)


def _tc_final_body(aggp_ref, aux_ref, cpt_ref, b3_ref, w4_ref, b4_ref,
                   out_ref, acc_ref):
  i = pl.program_id(0)
  a = aux_ref[...]
  so = lax.rsqrt(jnp.maximum(a[:, 0:1] + a[:, 1:2], 1.0))
  si = lax.rsqrt(jnp.maximum(a[:, 2:3] + a[:, 3:4], 1.0))
  cpt = cpt_ref[...]
  w_col = (cpt[:, 0:1] + cpt[:, 1:2]) * so
  agg = aggp_ref[0] + aggp_ref[1]
  h = jnp.maximum(agg * si + b3_ref[...], 0.0)
  part = jnp.sum(h * w_col, axis=0, keepdims=True)

  @pl.when(i == 0)
  def _():
    acc_ref[...] = part

  @pl.when(i > 0)
  def _():
    acc_ref[...] += part

  @pl.when(i == GRID - 1)
  def _():
    out_ref[...] = (
        jnp.dot(acc_ref[...], w4_ref[...], preferred_element_type=jnp.float32)
        / float(N) + b4_ref[...])


def _tc_final(aggp, aux, cpt, b3, w4, b4):
  return pl.pallas_call(
      _tc_final_body,
      grid=(GRID,),
      in_specs=[
          pl.BlockSpec((NC, R, F), lambda i: (0, i, 0)),
          pl.BlockSpec((R, 4), lambda i: (i, 0)),
          pl.BlockSpec((R, 2), lambda i: (i, 0)),
          pl.BlockSpec((1, F), lambda i: (0, 0)),
          pl.BlockSpec((F, C), lambda i: (0, 0)),
          pl.BlockSpec((1, C), lambda i: (0, 0)),
      ],
      out_specs=pl.BlockSpec((1, C), lambda i: (0, 0)),
      out_shape=jax.ShapeDtypeStruct((1, C), jnp.float32),
      scratch_shapes=[pltpu.VMEM((1, F), jnp.float32)],
  )(aggp, aux, cpt, b3.reshape(1, F), w4, b4.reshape(1, C))


def kernel(in_feat, edge_index, W1, b1, W2, b2, W3, b3, W4, b4):
  ei = edge_index.astype(jnp.int32).reshape(2 * E)

  h1u = _tc_mm(in_feat, W1)
  dop, dip = _sc_degrees(ei)
  dop = dop.reshape(NC, DEG_P)
  dip = dip.reshape(NC, DEG_P)
  aux = jnp.concatenate([dop, dip], axis=0)[:, :N].T

  h1p, sin_col = _tc_scale(h1u, aux)
  aggp1, cp = _sc_agg_c(h1p, ei, sin_col.reshape(N))
  h2p = _tc_mid(aggp1, aux, b1, W2)
  aggp2 = _sc_agg(h2p, ei)
  h3p = _tc_mid(aggp2, aux, b2, W3)
  aggp3 = _sc_agg(h3p, ei)
  cpt = cp.reshape(NC, DEG_P)[:, :N].T
  out = _tc_final(aggp3, aux, cpt, b3, W4, b4)
  return out.reshape(C)
